# baseline (speedup 1.0000x reference)
import jax
import jax.numpy as jnp
from jax import lax
from jax.experimental import pallas as pl
from jax.experimental.pallas import tpu as pltpu
from jax.experimental.pallas import tpu_sc as plsc

N = 2048
D = 256
K1 = 11
BLK = 256
GRID = N // BLK
EL = 32
AFLAT = N * N

NSUB = 16
ZCHUNK = 8192
ZITER = AFLAT // NSUB // ZCHUNK
EROWS = N * EL // 128
ERPS = EROWS // NSUB
SCAT_GROUP = 8


def _knn_edges_body(nodes_blk_ref, nodes_ref, tgt_ref, val_ref, nz_ref):
    i = pl.program_id(0)
    xb = nodes_blk_ref[...]
    xall = nodes_ref[...]
    sq_all = jnp.sum(xall * xall, axis=1)
    sq_blk = jnp.sum(xb * xb, axis=1)
    g = lax.dot_general(
        xb, xall, (((1,), (1,)), ((), ())),
        preferred_element_type=jnp.float32)
    s = sq_blk[:, None] + sq_all[None, :] - 2.0 * g
    s = jnp.maximum(s, 0.0)
    col = lax.broadcasted_iota(jnp.int32, (BLK, N), 1)
    row = lax.broadcasted_iota(jnp.int32, (BLK, N), 0) + i * BLK
    s = jnp.where(col == row, 0.0, s)
    big = jnp.float32(jnp.inf)

    lane = lax.broadcasted_iota(jnp.int32, (BLK, EL), 1)
    rowg = lax.broadcasted_iota(jnp.int32, (BLK, EL), 0) + i * BLK
    mn_off = jnp.min(jnp.where(col == row, big, s))

    @pl.when(jnp.exp(mn_off * -0.5) == 0.0)
    def _trivial():
        tgt_ref[...] = rowg * N + rowg
        val_ref[...] = jnp.ones((BLK, EL), jnp.float32)
        nz_ref[...] = jnp.zeros((1, 1, 128), jnp.float32)

    @pl.when(jnp.exp(mn_off * -0.5) != 0.0)
    def _general():
        work = s
        js, vs = [], []
        for _ in range(K1):
            m = jnp.min(work, axis=1, keepdims=True)
            is_min = work == m
            first = jnp.min(jnp.where(is_min, col, N), axis=1, keepdims=True)
            work = jnp.where(col == first, big, work)
            js.append(first)
            vs.append(jnp.exp(m * -0.5))

        tgt = rowg * N + jnp.broadcast_to(js[0], (BLK, EL))
        val = jnp.broadcast_to(vs[0], (BLK, EL))
        for t in range(K1):
            jt = jnp.broadcast_to(js[t], (BLK, EL))
            vt = jnp.broadcast_to(vs[t], (BLK, EL))
            tgt = jnp.where(lane == t, rowg * N + jt, tgt)
            tgt = jnp.where(lane == K1 + t, jt * N + rowg, tgt)
            val = jnp.where((lane == t) | (lane == K1 + t), vt, val)
        tgt_ref[...] = tgt
        val_ref[...] = val

        offd = jnp.zeros((BLK, 1), dtype=jnp.float32)
        for t in range(K1):
            offd = jnp.maximum(
                offd, jnp.where(js[t] == rowg[:, :1], 0.0, vs[t]))
        nz_ref[...] = jnp.broadcast_to(jnp.max(offd), (1, 1, 128))


def _scatter_sc(tgt_hbm, val_hbm, a_hbm, zbuf, tgtv, valv, sem):
    sid = lax.axis_index("s")

    def _zb(t, c):
        zbuf[pl.ds(t * 16, 16)] = jnp.zeros((16,), jnp.float32)
        return c
    lax.fori_loop(0, ZCHUNK // 16, _zb, 0, unroll=8)
    base = sid * (AFLAT // NSUB)
    handles = [
        pltpu.async_copy(zbuf, a_hbm.at[pl.ds(base + t * ZCHUNK, ZCHUNK)], sem)
        for t in range(ZITER)
    ]
    for h in handles:
        h.wait()
    plsc.subcore_barrier()

    rbase = sid * ERPS
    pltpu.sync_copy(tgt_hbm.at[pl.ds(rbase, ERPS)], tgtv)
    pltpu.sync_copy(val_hbm.at[pl.ds(rbase, ERPS)], valv)

    def _grp(gidx, c):
        hs = []
        for j in range(SCAT_GROUP):
            r = gidx * SCAT_GROUP + j
            hs.append(pltpu.async_copy(valv.at[r], a_hbm.at[tgtv.at[r]], sem))
        for h in hs:
            h.wait()
        return c
    lax.fori_loop(0, ERPS // SCAT_GROUP, _grp, 0)


def _deg_body(a_ref, deg_ref):
    deg_ref[...] = jnp.broadcast_to(
        jnp.sum(a_ref[...], axis=1)[:, None], (BLK, 128))


def _lap_body(a_ref, deg_ref, l_ref):
    i = pl.program_id(0)
    a = a_ref[...]
    deg = deg_ref[...]
    dinv = 1.0 / jnp.sqrt(deg)
    deg_r = deg_ref[pl.ds(i * BLK, BLK)]
    dinv_r = 1.0 / jnp.sqrt(deg_r)
    col = lax.broadcasted_iota(jnp.int32, (BLK, N), 1)
    row = lax.broadcasted_iota(jnp.int32, (BLK, N), 0) + i * BLK
    lmat = jnp.where(col == row, deg_r[:, None], 0.0) - a
    m1 = (dinv_r[:, None] * lmat) * dinv[None, :]
    m2 = (dinv[None, :] * lmat) * dinv_r[:, None]
    l_ref[...] = 0.5 * (m1 + m2)


def _dense_spectral(a):
    deg2d = pl.pallas_call(
        _deg_body,
        grid=(GRID,),
        in_specs=[pl.BlockSpec((BLK, N), lambda i: (i, 0))],
        out_specs=pl.BlockSpec((BLK, 128), lambda i: (i, 0)),
        out_shape=jax.ShapeDtypeStruct((N, 128), jnp.float32),
    )(a)
    deg = deg2d[:, 0]
    lsym = pl.pallas_call(
        _lap_body,
        grid=(GRID,),
        in_specs=[
            pl.BlockSpec((BLK, N), lambda i: (i, 0)),
            pl.BlockSpec((N,), lambda i: (0,)),
        ],
        out_specs=pl.BlockSpec((BLK, N), lambda i: (i, 0)),
        out_shape=jax.ShapeDtypeStruct((N, N), jnp.float32),
    )(a, deg)
    return jnp.linalg.eigh(lsym)[1]


def kernel(nodes, labels):
    del labels
    tgt, val, nz = pl.pallas_call(
        _knn_edges_body,
        grid=(GRID,),
        in_specs=[
            pl.BlockSpec((BLK, D), lambda i: (i, 0)),
            pl.BlockSpec((N, D), lambda i: (0, 0)),
        ],
        out_specs=[
            pl.BlockSpec((BLK, EL), lambda i: (i, 0)),
            pl.BlockSpec((BLK, EL), lambda i: (i, 0)),
            pl.BlockSpec((1, 1, 128), lambda i: (i, 0, 0)),
        ],
        out_shape=[
            jax.ShapeDtypeStruct((N, EL), jnp.int32),
            jax.ShapeDtypeStruct((N, EL), jnp.float32),
            jax.ShapeDtypeStruct((GRID, 1, 128), jnp.float32),
        ],
    )(nodes, nodes)

    mesh = plsc.VectorSubcoreMesh(
        core_axis_name="c", subcore_axis_name="s", num_cores=1)
    a_flat = pl.kernel(
        _scatter_sc,
        out_type=jax.ShapeDtypeStruct((AFLAT,), jnp.float32),
        mesh=mesh,
        scratch_types=[
            pltpu.VMEM((ZCHUNK,), jnp.float32),
            pltpu.VMEM((ERPS, 128), jnp.int32),
            pltpu.VMEM((ERPS, 128), jnp.float32),
            pltpu.SemaphoreType.DMA,
        ],
    )(tgt.reshape(EROWS, 128), val.reshape(EROWS, 128))
    a = a_flat.reshape(N, N)

    return lax.cond(
        jnp.any(nz != 0.0),
        _dense_spectral,
        lambda m: m,
        a,
    )


# speedup vs baseline: 3.5388x; 3.5388x over previous
import jax
import jax.numpy as jnp
from jax import lax
from jax.experimental import pallas as pl
from jax.experimental.pallas import tpu as pltpu
from jax.experimental.pallas import tpu_sc as plsc

N = 2048
D = 256
K1 = 11
BLK = 256
GRID = N // BLK
EL = 32
AFLAT = N * N

NSUB = 16
ZCHUNK = 8192
ZITER = AFLAT // NSUB // ZCHUNK
EROWS = N * EL // 128
ERPS = EROWS // NSUB
SCAT_GROUP = 8


def _knn_edges_body(nodes_blk_ref, nodes_ref, tgt_ref, val_ref, nz_ref):
    i = pl.program_id(0)
    xb = nodes_blk_ref[...]
    xall = nodes_ref[...]
    sq_all = jnp.sum(xall * xall, axis=1)
    sq_blk = jnp.sum(xb * xb, axis=1)
    g = lax.dot_general(
        xb, xall, (((1,), (1,)), ((), ())),
        preferred_element_type=jnp.float32)
    s = sq_blk[:, None] + sq_all[None, :] - 2.0 * g
    s = jnp.maximum(s, 0.0)
    col = lax.broadcasted_iota(jnp.int32, (BLK, N), 1)
    row = lax.broadcasted_iota(jnp.int32, (BLK, N), 0) + i * BLK
    s = jnp.where(col == row, 0.0, s)
    big = jnp.float32(jnp.inf)

    lane = lax.broadcasted_iota(jnp.int32, (BLK, EL), 1)
    rowg = lax.broadcasted_iota(jnp.int32, (BLK, EL), 0) + i * BLK
    mn_off = jnp.min(jnp.where(col == row, big, s))

    @pl.when(jnp.exp(mn_off * -0.5) == 0.0)
    def _trivial():
        tgt_ref[...] = rowg * N + rowg
        val_ref[...] = jnp.ones((BLK, EL), jnp.float32)
        nz_ref[...] = jnp.zeros((1, 1, 128), jnp.float32)

    @pl.when(jnp.exp(mn_off * -0.5) != 0.0)
    def _general():
        work = s
        js, vs = [], []
        for _ in range(K1):
            m = jnp.min(work, axis=1, keepdims=True)
            is_min = work == m
            first = jnp.min(jnp.where(is_min, col, N), axis=1, keepdims=True)
            work = jnp.where(col == first, big, work)
            js.append(first)
            vs.append(jnp.exp(m * -0.5))

        tgt = rowg * N + jnp.broadcast_to(js[0], (BLK, EL))
        val = jnp.broadcast_to(vs[0], (BLK, EL))
        for t in range(K1):
            jt = jnp.broadcast_to(js[t], (BLK, EL))
            vt = jnp.broadcast_to(vs[t], (BLK, EL))
            tgt = jnp.where(lane == t, rowg * N + jt, tgt)
            tgt = jnp.where(lane == K1 + t, jt * N + rowg, tgt)
            val = jnp.where((lane == t) | (lane == K1 + t), vt, val)
        tgt_ref[...] = tgt
        val_ref[...] = val

        offd = jnp.zeros((BLK, 1), dtype=jnp.float32)
        for t in range(K1):
            offd = jnp.maximum(
                offd, jnp.where(js[t] == rowg[:, :1], 0.0, vs[t]))
        nz_ref[...] = jnp.broadcast_to(jnp.max(offd), (1, 1, 128))


def _scatter_sc(tgt_hbm, val_hbm, a_hbm, zbuf, tgtv, valv, sem):
    sid = lax.axis_index("s")

    def _zb(t, c):
        zbuf[pl.ds(t * 16, 16)] = jnp.zeros((16,), jnp.float32)
        return c
    lax.fori_loop(0, ZCHUNK // 16, _zb, 0, unroll=8)
    base = sid * (AFLAT // NSUB)
    handles = [
        pltpu.async_copy(zbuf, a_hbm.at[pl.ds(base + t * ZCHUNK, ZCHUNK)], sem)
        for t in range(ZITER)
    ]
    for h in handles:
        h.wait()
    plsc.subcore_barrier()

    rbase = sid * ERPS
    pltpu.sync_copy(tgt_hbm.at[pl.ds(rbase, ERPS)], tgtv)
    pltpu.sync_copy(val_hbm.at[pl.ds(rbase, ERPS)], valv)

    def _grp(gidx, c):
        hs = []
        for j in range(SCAT_GROUP):
            r = gidx * SCAT_GROUP + j
            hs.append(pltpu.async_copy(valv.at[r], a_hbm.at[tgtv.at[r]], sem))
        for h in hs:
            h.wait()
        return c
    lax.fori_loop(0, ERPS // SCAT_GROUP, _grp, 0)


def _identity_sc(a_hbm, zbuf, idxv, valv, sem):
    sid = lax.axis_index("s")
    def _zb(t, c):
        zbuf[pl.ds(t * 16, 16)] = jnp.zeros((16,), jnp.float32)
        return c
    lax.fori_loop(0, ZCHUNK // 16, _zb, 0, unroll=8)
    base = sid * (AFLAT // NSUB)
    handles = [
        pltpu.async_copy(zbuf, a_hbm.at[pl.ds(base + t * ZCHUNK, ZCHUNK)], sem)
        for t in range(ZITER)
    ]
    rbase = sid * (N // NSUB)
    iot = lax.iota(jnp.int32, 16)
    for t in range(8):
        idxv[pl.ds(t * 16, 16)] = (rbase + t * 16 + iot) * (N + 1)
        valv[pl.ds(t * 16, 16)] = jnp.ones((16,), jnp.float32)
    for h in handles:
        h.wait()
    pltpu.async_copy(valv, a_hbm.at[idxv], sem).wait()


def _deg_body(a_ref, deg_ref):
    deg_ref[...] = jnp.broadcast_to(
        jnp.sum(a_ref[...], axis=1)[:, None], (BLK, 128))


def _lap_body(a_ref, deg_ref, l_ref):
    i = pl.program_id(0)
    a = a_ref[...]
    deg = deg_ref[...]
    dinv = 1.0 / jnp.sqrt(deg)
    deg_r = deg_ref[pl.ds(i * BLK, BLK)]
    dinv_r = 1.0 / jnp.sqrt(deg_r)
    col = lax.broadcasted_iota(jnp.int32, (BLK, N), 1)
    row = lax.broadcasted_iota(jnp.int32, (BLK, N), 0) + i * BLK
    lmat = jnp.where(col == row, deg_r[:, None], 0.0) - a
    m1 = (dinv_r[:, None] * lmat) * dinv[None, :]
    m2 = (dinv[None, :] * lmat) * dinv_r[:, None]
    l_ref[...] = 0.5 * (m1 + m2)


def _dense_spectral(a):
    deg2d = pl.pallas_call(
        _deg_body,
        grid=(GRID,),
        in_specs=[pl.BlockSpec((BLK, N), lambda i: (i, 0))],
        out_specs=pl.BlockSpec((BLK, 128), lambda i: (i, 0)),
        out_shape=jax.ShapeDtypeStruct((N, 128), jnp.float32),
    )(a)
    deg = deg2d[:, 0]
    lsym = pl.pallas_call(
        _lap_body,
        grid=(GRID,),
        in_specs=[
            pl.BlockSpec((BLK, N), lambda i: (i, 0)),
            pl.BlockSpec((N,), lambda i: (0,)),
        ],
        out_specs=pl.BlockSpec((BLK, N), lambda i: (i, 0)),
        out_shape=jax.ShapeDtypeStruct((N, N), jnp.float32),
    )(a, deg)
    return jnp.linalg.eigh(lsym)[1]


def kernel(nodes, labels):
    del labels
    tgt, val, nz = pl.pallas_call(
        _knn_edges_body,
        grid=(GRID,),
        in_specs=[
            pl.BlockSpec((BLK, D), lambda i: (i, 0)),
            pl.BlockSpec((N, D), lambda i: (0, 0)),
        ],
        out_specs=[
            pl.BlockSpec((BLK, EL), lambda i: (i, 0)),
            pl.BlockSpec((BLK, EL), lambda i: (i, 0)),
            pl.BlockSpec((1, 1, 128), lambda i: (i, 0, 0)),
        ],
        out_shape=[
            jax.ShapeDtypeStruct((N, EL), jnp.int32),
            jax.ShapeDtypeStruct((N, EL), jnp.float32),
            jax.ShapeDtypeStruct((GRID, 1, 128), jnp.float32),
        ],
    )(nodes, nodes)

    mesh = plsc.VectorSubcoreMesh(
        core_axis_name="c", subcore_axis_name="s", num_cores=1)

    def _general_h(ops):
        tgt_, val_ = ops
        a_flat = pl.kernel(
            _scatter_sc,
            out_type=jax.ShapeDtypeStruct((AFLAT,), jnp.float32),
            mesh=mesh,
            scratch_types=[
                pltpu.VMEM((ZCHUNK,), jnp.float32),
                pltpu.VMEM((ERPS, 128), jnp.int32),
                pltpu.VMEM((ERPS, 128), jnp.float32),
                pltpu.SemaphoreType.DMA,
            ],
        )(tgt_.reshape(EROWS, 128), val_.reshape(EROWS, 128))
        return _dense_spectral(a_flat.reshape(N, N))

    def _trivial_h(ops):
        del ops
        a_flat = pl.kernel(
            _identity_sc,
            out_type=jax.ShapeDtypeStruct((AFLAT,), jnp.float32),
            mesh=mesh,
            scratch_types=[
                pltpu.VMEM((ZCHUNK,), jnp.float32),
                pltpu.VMEM((128,), jnp.int32),
                pltpu.VMEM((128,), jnp.float32),
                pltpu.SemaphoreType.DMA,
            ],
        )()
        return a_flat.reshape(N, N)

    return lax.cond(jnp.any(nz != 0.0), _general_h, _trivial_h, (tgt, val))
